# Initial kernel scaffold; baseline (speedup 1.0000x reference)
import jax
import jax.numpy as jnp
from jax.experimental import pallas as pl


def kernel(x, edge_index, edge_attr, params):
    raise NotImplementedError("write your pallas kernel here")


# speedup vs baseline: 2.7748x; 2.7748x over previous
import functools

import jax
import jax.numpy as jnp
from jax import lax
from jax.experimental import pallas as pl
from jax.experimental.pallas import tpu as pltpu
from jax.experimental.pallas import tpu_sc as plsc

_N = 10000
_E = 320000
_D = 128
_LAYERS = 3

_NC = 2
_NS = 16
_NW = _NC * _NS
_EPW = _E // _NW
_C = 80
_NCHUNK = _EPW // _C
_NPAD = 10240
_RPT = _NPAD // _NS
_ZROWS = 128


def _edge_body(h_hbm, src_hbm, dst_hbm, ea_hbm, out_hbm,
               acc_sh, src_v, dst_v, rows_v, ea_v, stage_v, sem):
    c = lax.axis_index("c")
    s = lax.axis_index("s")
    wid = c * _NS + s

    zero16 = jnp.zeros((16,), jnp.float32)

    def zrow(r, carry):
        for g in range(8):
            stage_v[r, pl.ds(g * 16, 16)] = zero16
        return carry

    lax.fori_loop(0, _ZROWS, zrow, None)
    for b in range(_RPT // _ZROWS):
        pltpu.sync_copy(stage_v, acc_sh.at[pl.ds(s * _RPT + b * _ZROWS, _ZROWS)])
    plsc.subcore_barrier()

    base = wid * _EPW

    def chunk(i, carry):
        off = pl.multiple_of(base + i * _C, 8)
        pltpu.sync_copy(src_hbm.at[pl.ds(off, _C)], src_v)
        pltpu.sync_copy(dst_hbm.at[pl.ds(off, _C)], dst_v)
        pltpu.async_copy(h_hbm.at[src_v], rows_v, sem).wait()
        pltpu.sync_copy(ea_hbm.at[pl.ds(off, _C)], ea_v)

        def row(r, carry2):
            for g in range(8):
                sl = pl.ds(g * 16, 16)
                rows_v[r, sl] = jnp.maximum(rows_v[r, sl] + ea_v[r, sl], 0.0)
            return carry2

        lax.fori_loop(0, _C, row, None)
        pltpu.sync_copy(rows_v, acc_sh.at[dst_v], add=True)
        return carry

    lax.fori_loop(0, _NCHUNK, chunk, None)
    plsc.subcore_barrier()

    for b in range(_RPT // _ZROWS):
        rsl = pl.ds(s * _RPT + b * _ZROWS, _ZROWS)
        pltpu.sync_copy(acc_sh.at[rsl], stage_v)
        osl = pl.ds(c * _NPAD + s * _RPT + b * _ZROWS, _ZROWS)
        pltpu.sync_copy(stage_v, out_hbm.at[osl])


_edge_call = pl.kernel(
    _edge_body,
    out_type=jax.ShapeDtypeStruct((2 * _NPAD, _D), jnp.float32),
    mesh=plsc.VectorSubcoreMesh(
        core_axis_name="c", subcore_axis_name="s",
        num_cores=_NC, num_subcores=_NS),
    scratch_types=[
        pltpu.VMEM_SHARED((_NPAD, _D), jnp.float32),
        pltpu.VMEM((_C,), jnp.int32),
        pltpu.VMEM((_C,), jnp.int32),
        pltpu.VMEM((_C, _D), jnp.float32),
        pltpu.VMEM((_C, _D), jnp.float32),
        pltpu.VMEM((_ZROWS, _D), jnp.float32),
        pltpu.SemaphoreType.DMA,
    ],
)


def _ea_body(attr_ref, we_ref, be_ref, out_ref):
    out_ref[...] = (
        jnp.dot(attr_ref[...], we_ref[...], preferred_element_type=jnp.float32)
        + be_ref[...]
    )


_EB = 8000


@jax.jit
def _ea_call(edge_attr, we, be):
    return pl.pallas_call(
        _ea_body,
        grid=(_E // _EB,),
        in_specs=[
            pl.BlockSpec((_EB, 4), lambda i: (i, 0)),
            pl.BlockSpec((4, _D), lambda i: (0, 0)),
            pl.BlockSpec((1, _D), lambda i: (0, 0)),
        ],
        out_specs=pl.BlockSpec((_EB, _D), lambda i: (i, 0)),
        out_shape=jax.ShapeDtypeStruct((_E, _D), jnp.float32),
    )(edge_attr, we, be)


def _mlp_body(eps_ref, h_ref, a0_ref, a1_ref, w1_ref, b1_ref, w2_ref, b2_ref,
              g_ref, bt_ref, out_ref):
    h = h_ref[...]
    z = (1.0 + eps_ref[0]) * h + (a0_ref[0] + a1_ref[0])
    t = jnp.maximum(
        jnp.dot(z, w1_ref[...], preferred_element_type=jnp.float32)
        + b1_ref[...], 0.0)
    t = jnp.dot(t, w2_ref[...], preferred_element_type=jnp.float32) + b2_ref[...]
    t = jnp.maximum(t, 0.0) + h
    mu = jnp.mean(t, axis=-1, keepdims=True)
    zc = t - mu
    var = jnp.mean(zc * zc, axis=-1, keepdims=True)
    out_ref[...] = zc * lax.rsqrt(var + 1e-5) * g_ref[...] + bt_ref[...]


_RB = 2000


@jax.jit
def _mlp_call(eps, h, parts, w1, b1, w2, b2, g, bt):
    nblk = _N // _RB
    parts = parts.reshape(2, _NPAD, _D)
    return pl.pallas_call(
        _mlp_body,
        grid=(nblk,),
        in_specs=[
            pl.BlockSpec(memory_space=pltpu.SMEM),
            pl.BlockSpec((_RB, _D), lambda i: (i, 0)),
            pl.BlockSpec((1, _RB, _D), lambda i: (0, i, 0)),
            pl.BlockSpec((1, _RB, _D), lambda i: (1, i, 0)),
            pl.BlockSpec((_D, _D), lambda i: (0, 0)),
            pl.BlockSpec((1, _D), lambda i: (0, 0)),
            pl.BlockSpec((_D, _D), lambda i: (0, 0)),
            pl.BlockSpec((1, _D), lambda i: (0, 0)),
            pl.BlockSpec((1, _D), lambda i: (0, 0)),
            pl.BlockSpec((1, _D), lambda i: (0, 0)),
        ],
        out_specs=pl.BlockSpec((_RB, _D), lambda i: (i, 0)),
        out_shape=jax.ShapeDtypeStruct((_N, _D), jnp.float32),
    )(eps, h, parts, parts, w1, b1, w2, b2, g, bt)


@jax.jit
def kernel(x, edge_index, edge_attr, params):
    src = edge_index[0]
    dst = edge_index[1]
    ea = _ea_call(edge_attr, params['We'], params['be'].reshape(1, _D))
    h = x
    for l in range(_LAYERS):
        parts = _edge_call(h, src, dst, ea)
        h = _mlp_call(
            params['eps'][l].reshape(1),
            h, parts,
            params['W1'][l], params['b1'][l].reshape(1, _D),
            params['W2'][l], params['b2'][l].reshape(1, _D),
            params['ln_g'][l].reshape(1, _D), params['ln_b'][l].reshape(1, _D),
        )
    return h


# speedup vs baseline: 6.6861x; 2.4096x over previous
import functools

import jax
import jax.numpy as jnp
from jax import lax
from jax.experimental import pallas as pl
from jax.experimental.pallas import tpu as pltpu
from jax.experimental.pallas import tpu_sc as plsc

_N = 10000
_E = 320000
_D = 128
_LAYERS = 3

_NC = 2
_NS = 16
_NW = _NC * _NS
_EPW = _E // _NW
_C = 80
_NCHUNK = _EPW // _C
_NPAD = 10240
_RPT = _NPAD // _NS
_ZROWS = 32


def _edge_body(h_hbm, src_hbm, dst_hbm, ea_hbm, out_hbm,
               acc_sh, src4, dst4, rows2, ea2, stage_v,
               sem_src, sem_dst, sem_ea, sem_g, sem_sc):
    c = lax.axis_index("c")
    s = lax.axis_index("s")
    wid = c * _NS + s

    zero16 = jnp.zeros((16,), jnp.float32)

    def zrow(r, carry):
        for g in range(8):
            stage_v[r, pl.ds(g * 16, 16)] = zero16
        return carry

    lax.fori_loop(0, _ZROWS, zrow, None)
    for b in range(_RPT // _ZROWS):
        pltpu.sync_copy(stage_v, acc_sh.at[pl.ds(s * _RPT + b * _ZROWS, _ZROWS)])
    plsc.subcore_barrier()

    base = wid * _EPW

    def _off(ci):
        return pl.multiple_of(base + ci * _C, 8)

    def issue_idx(ci, p4):
        off = _off(ci)
        pltpu.async_copy(src_hbm.at[pl.ds(off, _C)], src4.at[p4], sem_src.at[p4])
        pltpu.async_copy(dst_hbm.at[pl.ds(off, _C)], dst4.at[p4], sem_dst.at[p4])

    def wait_idx(p4):
        pltpu.make_async_copy(
            src_hbm.at[pl.ds(0, _C)], src4.at[p4], sem_src.at[p4]).wait()
        pltpu.make_async_copy(
            dst_hbm.at[pl.ds(0, _C)], dst4.at[p4], sem_dst.at[p4]).wait()

    def issue_ea(ci, p2):
        pltpu.async_copy(ea_hbm.at[pl.ds(_off(ci), _C)], ea2.at[p2],
                         sem_ea.at[p2])

    def wait_ea(p2):
        pltpu.make_async_copy(
            ea_hbm.at[pl.ds(0, _C)], ea2.at[p2], sem_ea.at[p2]).wait()

    def issue_gather(pi4, pi2):
        pltpu.async_copy(h_hbm.at[src4.at[pi4]], rows2.at[pi2], sem_g.at[pi2])

    def wait_gather(p2, p4):
        pltpu.make_async_copy(
            h_hbm.at[src4.at[p4]], rows2.at[p2], sem_g.at[p2]).wait()

    def issue_scatter(p2, p4):
        pltpu.async_copy(rows2.at[p2], acc_sh.at[dst4.at[p4]], sem_sc.at[p2],
                         add=True)

    def wait_scatter(p2, p4):
        pltpu.make_async_copy(
            rows2.at[p2], acc_sh.at[dst4.at[p4]], sem_sc.at[p2]).wait()

    def compute(p2):
        rv = rows2.at[p2]
        ev = ea2.at[p2]

        def row(r, carry2):
            for g in range(8):
                sl = pl.ds(g * 16, 16)
                rv[r, sl] = jnp.maximum(rv[r, sl] + ev[r, sl], 0.0)
            return carry2

        lax.fori_loop(0, _C, row, None)

    def chunk_step(ci, p2, p4):
        pn2 = 1 - p2
        pn4 = (p4 + 1) % 4

        @pl.when((ci >= 1) & (ci + 1 < _NCHUNK))
        def _():
            wait_scatter(pn2, (p4 + 3) % 4)

        @pl.when(ci + 1 < _NCHUNK)
        def _():
            wait_idx(pn4)
            issue_gather(pn4, pn2)

        @pl.when(ci + 3 < _NCHUNK)
        def _():
            issue_idx(ci + 3, (p4 + 3) % 4)

        wait_gather(p2, p4)
        wait_ea(p2)
        compute(p2)
        issue_scatter(p2, p4)

        @pl.when(ci + 2 < _NCHUNK)
        def _():
            issue_ea(ci + 2, p2)

    issue_idx(0, 0)
    issue_idx(1, 1)
    issue_idx(2, 2)
    issue_ea(0, 0)
    issue_ea(1, 1)
    wait_idx(0)
    issue_gather(0, 0)

    def quad(q, carry):
        ci0 = q * 4
        for j in range(4):
            chunk_step(ci0 + j, j % 2, j)
        return carry

    lax.fori_loop(0, _NCHUNK // 4, quad, None)
    chunk_step(_NCHUNK - 1, 0, 0)
    wait_scatter(1, 3)
    wait_scatter(0, 0)
    plsc.subcore_barrier()

    for b in range(_RPT // _ZROWS):
        rsl = pl.ds(s * _RPT + b * _ZROWS, _ZROWS)
        pltpu.sync_copy(acc_sh.at[rsl], stage_v)
        osl = pl.ds(c * _NPAD + s * _RPT + b * _ZROWS, _ZROWS)
        pltpu.sync_copy(stage_v, out_hbm.at[osl])


_edge_call = pl.kernel(
    _edge_body,
    out_type=jax.ShapeDtypeStruct((2 * _NPAD, _D), jnp.float32),
    mesh=plsc.VectorSubcoreMesh(
        core_axis_name="c", subcore_axis_name="s",
        num_cores=_NC, num_subcores=_NS),
    scratch_types=[
        pltpu.VMEM_SHARED((_NPAD, _D), jnp.float32),
        pltpu.VMEM((4, _C), jnp.int32),
        pltpu.VMEM((4, _C), jnp.int32),
        pltpu.VMEM((2, _C, _D), jnp.float32),
        pltpu.VMEM((2, _C, _D), jnp.float32),
        pltpu.VMEM((_ZROWS, _D), jnp.float32),
        pltpu.SemaphoreType.DMA((4,)),
        pltpu.SemaphoreType.DMA((4,)),
        pltpu.SemaphoreType.DMA((2,)),
        pltpu.SemaphoreType.DMA((2,)),
        pltpu.SemaphoreType.DMA((2,)),
    ],
)


def _ea_body(attr_ref, we_ref, be_ref, out_ref):
    out_ref[...] = (
        jnp.dot(attr_ref[...], we_ref[...], preferred_element_type=jnp.float32)
        + be_ref[...]
    )


_EB = 8000


@jax.jit
def _ea_call(edge_attr, we, be):
    return pl.pallas_call(
        _ea_body,
        grid=(_E // _EB,),
        in_specs=[
            pl.BlockSpec((_EB, 4), lambda i: (i, 0)),
            pl.BlockSpec((4, _D), lambda i: (0, 0)),
            pl.BlockSpec((1, _D), lambda i: (0, 0)),
        ],
        out_specs=pl.BlockSpec((_EB, _D), lambda i: (i, 0)),
        out_shape=jax.ShapeDtypeStruct((_E, _D), jnp.float32),
    )(edge_attr, we, be)


def _mlp_body(eps_ref, h_ref, a0_ref, a1_ref, w1_ref, b1_ref, w2_ref, b2_ref,
              g_ref, bt_ref, out_ref):
    h = h_ref[...]
    z = (1.0 + eps_ref[0]) * h + (a0_ref[0] + a1_ref[0])
    t = jnp.maximum(
        jnp.dot(z, w1_ref[...], preferred_element_type=jnp.float32)
        + b1_ref[...], 0.0)
    t = jnp.dot(t, w2_ref[...], preferred_element_type=jnp.float32) + b2_ref[...]
    t = jnp.maximum(t, 0.0) + h
    mu = jnp.mean(t, axis=-1, keepdims=True)
    zc = t - mu
    var = jnp.mean(zc * zc, axis=-1, keepdims=True)
    out_ref[...] = zc * lax.rsqrt(var + 1e-5) * g_ref[...] + bt_ref[...]


_RB = 2000


@jax.jit
def _mlp_call(eps, h, parts, w1, b1, w2, b2, g, bt):
    nblk = _N // _RB
    parts = parts.reshape(2, _NPAD, _D)
    return pl.pallas_call(
        _mlp_body,
        grid=(nblk,),
        in_specs=[
            pl.BlockSpec(memory_space=pltpu.SMEM),
            pl.BlockSpec((_RB, _D), lambda i: (i, 0)),
            pl.BlockSpec((1, _RB, _D), lambda i: (0, i, 0)),
            pl.BlockSpec((1, _RB, _D), lambda i: (1, i, 0)),
            pl.BlockSpec((_D, _D), lambda i: (0, 0)),
            pl.BlockSpec((1, _D), lambda i: (0, 0)),
            pl.BlockSpec((_D, _D), lambda i: (0, 0)),
            pl.BlockSpec((1, _D), lambda i: (0, 0)),
            pl.BlockSpec((1, _D), lambda i: (0, 0)),
            pl.BlockSpec((1, _D), lambda i: (0, 0)),
        ],
        out_specs=pl.BlockSpec((_RB, _D), lambda i: (i, 0)),
        out_shape=jax.ShapeDtypeStruct((_N, _D), jnp.float32),
    )(eps, h, parts, parts, w1, b1, w2, b2, g, bt)


@jax.jit
def kernel(x, edge_index, edge_attr, params):
    src = edge_index[0]
    dst = edge_index[1]
    ea = _ea_call(edge_attr, params['We'], params['be'].reshape(1, _D))
    h = x
    for l in range(_LAYERS):
        parts = _edge_call(h, src, dst, ea)
        h = _mlp_call(
            params['eps'][l].reshape(1),
            h, parts,
            params['W1'][l], params['b1'][l].reshape(1, _D),
            params['W2'][l], params['b2'][l].reshape(1, _D),
            params['ln_g'][l].reshape(1, _D), params['ln_b'][l].reshape(1, _D),
        )
    return h
